# baseline (speedup 1.0000x reference)
import functools

import jax
import jax.numpy as jnp
from jax import lax
from jax.experimental import pallas as pl
from jax.experimental.pallas import tpu as pltpu
from jax.experimental.pallas import tpu_sc as plsc

B = 16384
NF = 32
NV = 1000000
NC = 2
NS = 16
NW = NC * NS
BPW = B // NW
L = 16
CHUNK = 128
NCH = BPW // CHUNK


def _make_sc_call():
    mesh = plsc.VectorSubcoreMesh(core_axis_name="c", subcore_axis_name="s")

    @functools.partial(
        pl.kernel,
        mesh=mesh,
        compiler_params=pltpu.CompilerParams(
            needs_layout_passes=False, use_tc_tiling_on_sc=False),
        out_type=jax.ShapeDtypeStruct((B,), jnp.float32),
        scratch_types=[
            pltpu.VMEM((NCH, CHUNK), jnp.int32),
            pltpu.VMEM((NCH, CHUNK), jnp.int32),
            pltpu.VMEM((NF, BPW), jnp.float32),
            pltpu.VMEM((NF, BPW), jnp.float32),
            pltpu.VMEM((BPW,), jnp.float32),
            pltpu.SemaphoreType.DMA,
            pltpu.SemaphoreType.DMA,
        ],
    )
    def sc_kernel(users_hbm, items_hbm, ut_hbm, it_hbm, out_hbm,
                  uidx_v, iidx_v, uval_v, ival_v, out_v,
                  sem_u, sem_i):
        wid = lax.axis_index("s") * NC + lax.axis_index("c")
        base = wid * BPW

        pltpu.sync_copy(users_hbm.at[wid], uidx_v)
        pltpu.sync_copy(items_hbm.at[wid], iidx_v)

        copies = []
        for k in range(NF):
            for c in range(NCH):
                copies.append(pltpu.async_copy(
                    ut_hbm.at[k].at[uidx_v.at[c]],
                    uval_v.at[k].at[pl.ds(c * CHUNK, CHUNK)], sem_u))
                copies.append(pltpu.async_copy(
                    it_hbm.at[k].at[iidx_v.at[c]],
                    ival_v.at[k].at[pl.ds(c * CHUNK, CHUNK)], sem_i))
        for cp in copies:
            cp.wait()

        def dot_group(g, carry):
            acc = jnp.zeros((L,), jnp.float32)
            for k in range(NF):
                acc = acc + (uval_v[k, pl.ds(g * L, L)]
                             * ival_v[k, pl.ds(g * L, L)])
            out_v[pl.ds(g * L, L)] = acc
            return carry

        lax.fori_loop(0, BPW // L, dot_group, 0)

        pltpu.sync_copy(out_v, out_hbm.at[pl.ds(base, BPW)])

    return sc_kernel


_sc_call = _make_sc_call()


@jax.jit
def kernel(data, user_factors, item_factors):
    data = data.astype(jnp.int32)
    users = data[:, 0].reshape(NW, NCH, CHUNK)
    items = data[:, 1].reshape(NW, NCH, CHUNK)
    return _sc_call(users, items, user_factors.T, item_factors.T)


# speedup vs baseline: 5.7110x; 5.7110x over previous
import functools

import jax
import jax.numpy as jnp
from jax import lax
from jax.experimental import pallas as pl
from jax.experimental.pallas import tpu as pltpu
from jax.experimental.pallas import tpu_sc as plsc

B = 16384
NF = 32
NC = 2
NS = 16
NW = NC * NS
BPW = B // NW
L = 16


def _make_sc_call():
    mesh = plsc.VectorSubcoreMesh(core_axis_name="c", subcore_axis_name="s")

    @functools.partial(
        pl.kernel,
        mesh=mesh,
        compiler_params=pltpu.CompilerParams(
            needs_layout_passes=False, use_tc_tiling_on_sc=False),
        out_type=jax.ShapeDtypeStruct((B,), jnp.float32),
        scratch_types=[
            pltpu.VMEM((BPW // 128, 128), jnp.int32),
            pltpu.VMEM((BPW // 128, 128), jnp.int32),
            pltpu.VMEM((BPW, NF), jnp.float32),
            pltpu.VMEM((BPW, NF), jnp.float32),
            pltpu.VMEM((BPW + L,), jnp.float32),
            pltpu.SemaphoreType.DMA,
            pltpu.SemaphoreType.DMA,
        ],
    )
    def sc_kernel(users_hbm, items_hbm, user_hbm, item_hbm, out_hbm,
                  uidx_v, iidx_v, urows_v, irows_v, out_v,
                  sem_u, sem_i):
        wid = lax.axis_index("s") * NC + lax.axis_index("c")
        base = wid * BPW
        nch = BPW // 128

        pltpu.sync_copy(users_hbm.at[wid], uidx_v)
        pltpu.sync_copy(items_hbm.at[wid], iidx_v)

        copies = []
        for k in range(nch):
            copies.append(pltpu.async_copy(
                user_hbm.at[uidx_v.at[k]],
                urows_v.at[pl.ds(k * 128, 128)], sem_u))
            copies.append(pltpu.async_copy(
                item_hbm.at[iidx_v.at[k]],
                irows_v.at[pl.ds(k * 128, 128)], sem_i))
        for cp in copies:
            cp.wait()

        lane = lax.iota(jnp.int32, L)
        last_mask = lane == (L - 1)

        def dot_chunk(c, carry):
            for j in range(L):
                r = c * L + j
                u0 = urows_v[r, pl.ds(0, L)]
                u1 = urows_v[r, pl.ds(L, L)]
                v0 = irows_v[r, pl.ds(0, L)]
                v1 = irows_v[r, pl.ds(L, L)]
                tot = plsc.cumsum(u0 * v0 + u1 * v1)
                plsc.store_compressed(out_v.at[pl.ds(r, L)], tot,
                                      mask=last_mask)
            return carry

        lax.fori_loop(0, BPW // L, dot_chunk, 0)

        pltpu.sync_copy(out_v.at[pl.ds(0, BPW)], out_hbm.at[pl.ds(base, BPW)])

    return sc_kernel


_sc_call = _make_sc_call()


@jax.jit
def kernel(data, user_factors, item_factors):
    data = data.astype(jnp.int32)
    users = data[:, 0].reshape(NW, BPW // 128, 128)
    items = data[:, 1].reshape(NW, BPW // 128, 128)
    return _sc_call(users, items, user_factors, item_factors)
